# baseline (speedup 1.0000x reference)
import functools
import jax
import jax.numpy as jnp
from jax import lax
from jax.experimental import pallas as pl
from jax.experimental.pallas import tpu as pltpu
from jax.experimental.pallas import tpu_sc as plsc

D = 64
DP = 128
SCALE = 8.0
NC, NS, L = 2, 16, 16
NW = NC * NS
NB = 200
BLK = 128
NBUF = 4
LOOKAHEAD = 2

_mesh = plsc.VectorSubcoreMesh(core_axis_name="c", subcore_axis_name="s")


@functools.partial(
    pl.kernel,
    mesh=_mesh,
    out_type=jax.ShapeDtypeStruct((NB, NW, BLK // 2, DP), jnp.float32),
    scratch_types=[
        [pltpu.VMEM((BLK,), jnp.int32) for _ in range(NBUF)],
        [pltpu.VMEM((BLK, DP), jnp.float32) for _ in range(NBUF)],
        [pltpu.VMEM((BLK // 2, DP), jnp.float32) for _ in range(NBUF)],
        [pltpu.SemaphoreType.DMA for _ in range(NBUF)],
        [pltpu.SemaphoreType.DMA for _ in range(NBUF)],
        [pltpu.SemaphoreType.DMA for _ in range(NBUF)],
    ],
    compiler_params=pltpu.CompilerParams(
        use_tc_tiling_on_sc=False, needs_layout_passes=False
    ),
)
def _emb_gather(tok_hbm, table_hbm, out_hbm, ibuf, gbuf, wbuf, isem, gsem, wsem):
    wid = lax.axis_index("s") * NC + lax.axis_index("c")

    def idx_load(g, b, use_sem):
        ts = lax.div(g, 8)
        si = lax.rem(g, 8)
        if use_sem:
            pltpu.async_copy(tok_hbm.at[ts, wid, si], ibuf[b], isem[b])
        else:
            pltpu.sync_copy(tok_hbm.at[ts, wid, si], ibuf[b])

    def gather_wait(b):
        pltpu.make_async_copy(table_hbm.at[pl.ds(0, BLK)], gbuf[b], gsem[b]).wait()

    def write_wait(b):
        pltpu.make_async_copy(
            table_hbm.at[pl.ds(0, BLK // 2)], wbuf[b], wsem[b]
        ).wait()

    def idx_wait(b):
        pltpu.make_async_copy(tok_hbm.at[0, 0, 0], ibuf[b], isem[b]).wait()

    idx_load(0, 0, False)
    idx_load(1, 1, False)
    idx_load(2, 2, True)
    idx_load(3, 3, True)
    pltpu.async_copy(table_hbm.at[ibuf[0]], gbuf[0], gsem[0])
    pltpu.async_copy(table_hbm.at[ibuf[1]], gbuf[1], gsem[1])

    def outer(i, carry):
        gbase = i * NBUF
        for b in range(NBUF):
            g = gbase + b
            gather_wait(b)

            @pl.when(g + NBUF < NB)
            def _():
                idx_load(g + NBUF, b, True)

            @pl.when(g >= NBUF)
            def _():
                write_wait(b)

            def pair_body(j, c2):
                for h in range(2):
                    for k in range(D // L):
                        v = gbuf[b][2 * j + h, pl.ds(k * L, L)] * SCALE
                        wbuf[b][j, pl.ds(h * D + k * L, L)] = v
                return c2

            lax.fori_loop(0, BLK // 2, pair_body, 0, unroll=2)

            pltpu.async_copy(wbuf[b], out_hbm.at[g, wid], wsem[b])

            g2 = g + LOOKAHEAD
            b2 = (b + LOOKAHEAD) % NBUF

            @pl.when(g2 < NB)
            def _():
                idx_wait(b2)
                pltpu.async_copy(table_hbm.at[ibuf[b2]], gbuf[b2], gsem[b2])

        return carry

    lax.fori_loop(0, NB // NBUF, outer, 0)
    for b in range(NBUF):
        write_wait(b)


def _tc_transpose_block(x_ref, o_ref):
    x = x_ref[:, 0]
    xt = jnp.swapaxes(x, 1, 2)
    a = xt[:, 0:64, :]
    b = xt[:, 64:128, :]
    y = jnp.stack([a, b], axis=-1).reshape(4, 64, 128)
    o_ref[:, :, 0] = y.reshape(4, 8, 8, 128)


def kernel(tokens, table):
    tok_phys = tokens.T.reshape(25, 8, NW, BLK).transpose(0, 2, 1, 3)
    table_pad = jnp.pad(table, ((0, 0), (0, D)))
    y = _emb_gather(tok_phys, table_pad)

    z = pl.pallas_call(
        _tc_transpose_block,
        grid=(NB // 4, NW),
        in_specs=[
            pl.BlockSpec((4, 1, BLK // 2, DP), lambda s, tb: (s, tb, 0, 0))
        ],
        out_specs=pl.BlockSpec((4, 8, 1, 8, BLK), lambda s, tb: (s, 0, tb, 0, 0)),
        out_shape=jax.ShapeDtypeStruct((NB, 8, NW, 8, BLK), jnp.float32),
    )(y)
    return z.transpose(2, 4, 0, 1, 3).reshape(tokens.shape[0], tokens.shape[1], D)


# speedup vs baseline: 9.5647x; 9.5647x over previous
import functools
import jax
import jax.numpy as jnp
from jax import lax
from jax.experimental import pallas as pl
from jax.experimental.pallas import tpu as pltpu
from jax.experimental.pallas import tpu_sc as plsc

D = 64
SCALE = 8.0
NC, NS, L = 2, 16, 16
NW = NC * NS
B = 4096 * 200
BPW = B // NW
C = 256
NCHUNK = BPW // C
NBUF = 4
LOOKAHEAD = 2

_mesh = plsc.VectorSubcoreMesh(core_axis_name="c", subcore_axis_name="s")


@functools.partial(
    pl.kernel,
    mesh=_mesh,
    out_type=jax.ShapeDtypeStruct((B, D), jnp.float32),
    scratch_types=[
        pltpu.VMEM((NCHUNK, C), jnp.int32),
        [pltpu.VMEM((C, D), jnp.float32) for _ in range(NBUF)],
        [pltpu.SemaphoreType.DMA for _ in range(NBUF)],
        [pltpu.SemaphoreType.DMA for _ in range(NBUF)],
    ],
    compiler_params=pltpu.CompilerParams(use_tc_tiling_on_sc=False),
)
def _emb_lookup(idx_hbm, table_hbm, out_hbm, idx_all, rows, gsem, wsem):
    wid = lax.axis_index("s") * NC + lax.axis_index("c")
    base = wid * BPW
    pltpu.sync_copy(idx_hbm.at[wid], idx_all)

    def gather_wait(b):
        pltpu.make_async_copy(table_hbm.at[pl.ds(0, C)], rows[b], gsem[b]).wait()

    def write_wait(b):
        pltpu.make_async_copy(table_hbm.at[pl.ds(0, C)], rows[b], wsem[b]).wait()

    for g in range(LOOKAHEAD):
        pltpu.async_copy(table_hbm.at[idx_all.at[g]], rows[g], gsem[g])

    def outer(i, carry):
        gbase = i * NBUF
        for b in range(NBUF):
            g = gbase + b
            gather_wait(b)

            def row_body(r, c2):
                for j in range(D // L):
                    rows[b][r, pl.ds(j * L, L)] = rows[b][r, pl.ds(j * L, L)] * SCALE
                return c2

            lax.fori_loop(0, C, row_body, 0, unroll=4)
            pltpu.async_copy(rows[b], out_hbm.at[pl.ds(base + g * C, C)], wsem[b])

            g2 = g + LOOKAHEAD
            b2 = (b + LOOKAHEAD) % NBUF

            @pl.when(g2 < NCHUNK)
            def _():
                @pl.when(g2 >= NBUF)
                def _():
                    write_wait(b2)

                pltpu.async_copy(table_hbm.at[idx_all.at[g2]], rows[b2], gsem[b2])

        return carry

    lax.fori_loop(0, NCHUNK // NBUF, outer, 0)
    for b in range(NBUF):
        write_wait(b)


def kernel(tokens, table):
    idx = tokens.astype(jnp.int32).reshape(NW, NCHUNK, C)
    out = _emb_lookup(idx, table)
    return out.reshape(tokens.shape[0], tokens.shape[1], D)
